# baseline (speedup 1.0000x reference)
import functools

import jax
import jax.numpy as jnp
from jax import lax
from jax.experimental import pallas as pl
from jax.experimental.pallas import tpu as pltpu
from jax.experimental.pallas import tpu_sc as plsc

N_ROWS = 1_000_000
D = 32
F = 26
BATCH = 16384
B = BATCH * F
NC, NS = 2, 16
NW = NC * NS
UR = 512
UNITS = B // UR
UPW = UNITS // NW
BPW = UPW * UR

_mesh = plsc.VectorSubcoreMesh(core_axis_name="c", subcore_axis_name="s")


@functools.partial(
    pl.kernel,
    mesh=_mesh,
    out_type=jax.ShapeDtypeStruct((F, D // 8, BATCH // 128, 8, 128),
                                  jnp.float32),
    scratch_types=[
        pltpu.VMEM((BPW,), jnp.int32),
        pltpu.VMEM((2, UR, D), jnp.float32),
        pltpu.VMEM((2, 4, 4, 8, 128), jnp.float32),
        pltpu.SemaphoreType.DMA,
        pltpu.SemaphoreType.DMA,
        pltpu.SemaphoreType.DMA,
        pltpu.SemaphoreType.DMA,
    ],
    compiler_params=pltpu.CompilerParams(
        use_tc_tiling_on_sc=False, needs_layout_passes=False),
)
def _gather_kernel(table_hbm, idx_hbm, out_hbm, idx_v, rows_v, tiles_v,
                   gsem0, gsem1, wsem0, wsem1):
    wid = lax.axis_index("s") * NC + lax.axis_index("c")
    base = wid * BPW
    pltpu.sync_copy(idx_hbm.at[pl.ds(base, BPW)], idx_v)

    gsems = (gsem0, gsem1)
    wsems = (wsem0, wsem1)
    lane = lax.iota(jnp.int32, 16)
    gvecs = [g * 16 + lane for g in range(8)]
    g0 = wid * UPW

    def start_gather(t, par):
        pltpu.async_copy(
            table_hbm.at[idx_v.at[pl.ds(t * UR, UR)]],
            rows_v.at[par], gsems[par])

    def wait_gather(par):
        pltpu.make_async_copy(
            table_hbm.at[pl.ds(0, UR)], rows_v.at[par], gsems[par]).wait()

    def drain_writes(par):
        for fb in range(4):
            pltpu.make_async_copy(
                tiles_v.at[par, fb], out_hbm.at[0, 0, pl.ds(0, 4)],
                wsems[par]).wait()

    def transpose_unit(par):
        rows = rows_v.at[par]
        tiles = tiles_v.at[par]

        def body(m, carry):
            fb = m // 32
            bbl = lax.rem(m // 8, 4)
            fr = lax.rem(m, 8)
            cvec = jnp.full((16,), fb * 8 + fr, jnp.int32)
            rbase = bbl * 128
            for g in range(8):
                val = plsc.load_gather(rows, [rbase + gvecs[g], cvec])
                tiles[fb, bbl, fr, pl.ds(g * 16, 16)] = val
            return carry

        lax.fori_loop(0, 128, body, 0)

    start_gather(0, 0)
    start_gather(1, 1)

    def unit_pair(tt, carry):
        for par in range(2):
            t = tt * 2 + par
            wait_gather(par)

            @pl.when(tt > 0)
            def _():
                drain_writes(par)

            transpose_unit(par)

            @pl.when(t + 2 < UPW)
            def _():
                start_gather(t + 2, par)

            gu = g0 + t
            j = gu // 32
            bbq = lax.rem(gu, 32)
            for fb in range(4):
                pltpu.async_copy(
                    tiles_v.at[par, fb],
                    out_hbm.at[j, fb, pl.ds(bbq * 4, 4)],
                    wsems[par])
        return carry

    lax.fori_loop(0, UPW // 2, unit_pair, 0)
    for par in range(2):
        drain_writes(par)


def kernel(x, weights):
    idx_fm = x.T.reshape(-1).astype(jnp.int32)
    out5 = _gather_kernel(weights, idx_fm)
    out = out5.transpose(2, 4, 0, 1, 3).reshape(BATCH, F, D)
    return out


# speedup vs baseline: 1.3476x; 1.3476x over previous
import functools

import jax
import jax.numpy as jnp
from jax import lax
from jax.experimental import pallas as pl
from jax.experimental.pallas import tpu as pltpu
from jax.experimental.pallas import tpu_sc as plsc

N_ROWS = 1_000_000
D = 32
F = 26
BATCH = 16384
B = BATCH * F
NC, NS = 2, 16
NW = NC * NS
UR = 512
UNITS = B // UR
UPW = UNITS // NW
BPW = UPW * UR

_mesh = plsc.VectorSubcoreMesh(core_axis_name="c", subcore_axis_name="s")


@functools.partial(
    pl.kernel,
    mesh=_mesh,
    out_type=jax.ShapeDtypeStruct((F, D // 8, BATCH // 128, 8, 128),
                                  jnp.float32),
    scratch_types=[
        pltpu.VMEM((BPW,), jnp.int32),
        pltpu.VMEM((2, UR, D), jnp.float32),
        pltpu.VMEM((2, 4, 4, 8, 131), jnp.float32),
        pltpu.SemaphoreType.DMA,
        pltpu.SemaphoreType.DMA,
        pltpu.SemaphoreType.DMA,
        pltpu.SemaphoreType.DMA,
    ],
    compiler_params=pltpu.CompilerParams(
        use_tc_tiling_on_sc=False, needs_layout_passes=False),
)
def _gather_kernel(table_hbm, idx_hbm, out_hbm, idx_v, rows_v, tiles_v,
                   gsem0, gsem1, wsem0, wsem1):
    wid = lax.axis_index("s") * NC + lax.axis_index("c")
    base = wid * BPW
    pltpu.sync_copy(idx_hbm.at[pl.ds(base, BPW)], idx_v)

    gsems = (gsem0, gsem1)
    wsems = (wsem0, wsem1)
    lane = lax.iota(jnp.int32, 16)
    gvecs = [g * 16 + lane for g in range(8)]
    g0 = wid * UPW

    def start_gather(t, par):
        pltpu.async_copy(
            table_hbm.at[idx_v.at[pl.ds(t * UR, UR)]],
            rows_v.at[par], gsems[par])

    def wait_gather(par):
        pltpu.make_async_copy(
            table_hbm.at[pl.ds(0, UR)], rows_v.at[par], gsems[par]).wait()

    def drain_writes(par):
        for fb in range(4):
            pltpu.make_async_copy(
                tiles_v.at[par, :, fb, :, pl.ds(0, 128)],
                out_hbm.at[0, 0, pl.ds(0, 4)], wsems[par]).wait()

    frvec = lax.rem(lane, 8)
    fbhalf = lane // 8

    def transpose_unit(par):
        rows = rows_v.at[par]
        tiles = tiles_v.at[par]
        fbvecs = [fbhalf + 2 * g for g in range(2)]

        def body(r4, carry):
            r0 = r4 * 4
            bblvec = jnp.full((16,), r0 // 128, jnp.int32)
            bc0vec = jnp.full((16,), lax.rem(r0, 128), jnp.int32)
            for dr in range(4):
                r = r0 + dr
                bcvec = bc0vec + dr if dr else bc0vec
                for g in range(2):
                    val = rows[r, pl.ds(g * 16, 16)]
                    plsc.store_scatter(
                        tiles, [bblvec, fbvecs[g], frvec, bcvec], val)
            return carry

        lax.fori_loop(0, UR // 4, body, 0)

    start_gather(0, 0)
    start_gather(1, 1)

    def unit_pair(tt, carry):
        for par in range(2):
            t = tt * 2 + par
            wait_gather(par)

            @pl.when(tt > 0)
            def _():
                drain_writes(par)

            transpose_unit(par)

            @pl.when(t + 2 < UPW)
            def _():
                start_gather(t + 2, par)

            gu = g0 + t
            j = gu // 32
            bbq = lax.rem(gu, 32)
            for fb in range(4):
                pltpu.async_copy(
                    tiles_v.at[par, :, fb, :, pl.ds(0, 128)],
                    out_hbm.at[j, fb, pl.ds(bbq * 4, 4)],
                    wsems[par])
        return carry

    lax.fori_loop(0, UPW // 2, unit_pair, 0)
    for par in range(2):
        drain_writes(par)


def kernel(x, weights):
    idx_fm = x.T.reshape(-1).astype(jnp.int32)
    out5 = _gather_kernel(weights, idx_fm)
    out = out5.transpose(2, 4, 0, 1, 3).reshape(BATCH, F, D)
    return out
